# Initial kernel scaffold; baseline (speedup 1.0000x reference)
import jax
import jax.numpy as jnp
from jax.experimental import pallas as pl


def kernel(x, table):
    raise NotImplementedError("write your pallas kernel here")


# speedup vs baseline: 3.7327x; 3.7327x over previous
import functools

import jax
import jax.numpy as jnp
from jax import lax
from jax.experimental import pallas as pl
from jax.experimental.pallas import tpu as pltpu
from jax.experimental.pallas import tpu_sc as plsc

EMBED = 64
_info = plsc.get_sparse_core_info()
NC, NS = _info.num_cores, _info.num_subcores
NW = NC * NS


def _make_sc_lookup(B: int, C: int):
    assert B % (NW * C) == 0
    b_per_w = B // NW
    chunks = b_per_w // C
    mesh = plsc.VectorSubcoreMesh(core_axis_name="c", subcore_axis_name="s")

    @functools.partial(
        pl.kernel,
        out_type=jax.ShapeDtypeStruct((B, EMBED), jnp.float32),
        mesh=mesh,
        scratch_types=[
            pltpu.VMEM((C,), jnp.int32),
            pltpu.VMEM((C, EMBED), jnp.float32),
            pltpu.SemaphoreType.DMA,
        ],
        compiler_params=pltpu.CompilerParams(use_tc_tiling_on_sc=False),
    )
    def sc_lookup(x_hbm, table_hbm, out_hbm, idx_v, rows_v, sem):
        wid = lax.axis_index("s") * NC + lax.axis_index("c")
        base0 = wid * b_per_w

        @pl.loop(0, chunks)
        def _chunk(c):
            base = base0 + c * C
            pltpu.sync_copy(x_hbm.at[pl.ds(base, C)], idx_v)
            pltpu.async_copy(table_hbm.at[idx_v], rows_v, sem).wait()
            pltpu.sync_copy(rows_v, out_hbm.at[pl.ds(base, C)])

    return sc_lookup


def kernel(x, table):
    lead = x.shape[:-1]
    k = x.shape[-1]
    B = 1
    for s in x.shape:
        B *= s
    xf = x.reshape(B).astype(jnp.int32)
    out = _make_sc_lookup(B, 512)(xf, table)
    return out.reshape(lead + (k * EMBED,))


# speedup vs baseline: 6.4829x; 1.7368x over previous
import functools

import jax
import jax.numpy as jnp
from jax import lax
from jax.experimental import pallas as pl
from jax.experimental.pallas import tpu as pltpu
from jax.experimental.pallas import tpu_sc as plsc

EMBED = 64
N_BINS = 256
_info = plsc.get_sparse_core_info()
NC, NS = _info.num_cores, _info.num_subcores
NW = NC * NS


def _make_sc_lookup(B: int, C: int):
    assert B % (NW * C) == 0
    b_per_w = B // NW
    chunks = b_per_w // C
    mesh = plsc.VectorSubcoreMesh(core_axis_name="c", subcore_axis_name="s")

    @functools.partial(
        pl.kernel,
        out_type=jax.ShapeDtypeStruct((B, EMBED), jnp.float32),
        mesh=mesh,
        scratch_types=[
            pltpu.VMEM_SHARED((N_BINS, EMBED), jnp.float32),
            pltpu.VMEM((2, C), jnp.int32),
            pltpu.VMEM((2, C, EMBED), jnp.float32),
            pltpu.SemaphoreType.DMA((2,)),
            pltpu.SemaphoreType.DMA((2,)),
            pltpu.SemaphoreType.DMA((2,)),
        ],
        compiler_params=pltpu.CompilerParams(use_tc_tiling_on_sc=False),
    )
    def sc_lookup(x_hbm, table_hbm, out_hbm, table_s, idx_v, rows_v,
                  sem_idx, sem_g, sem_out):
        sid = lax.axis_index("s")
        wid = sid * NC + lax.axis_index("c")
        base0 = wid * b_per_w

        @pl.when(sid == 0)
        def _stage():
            pltpu.sync_copy(table_hbm, table_s)

        plsc.subcore_barrier()

        pltpu.async_copy(x_hbm.at[pl.ds(base0, C)], idx_v.at[0],
                         sem_idx.at[0])

        @pl.loop(0, chunks)
        def _chunk(c):
            b = c % 2
            nb = 1 - b

            @pl.when(c + 1 < chunks)
            def _prefetch():
                nbase = base0 + (c + 1) * C
                pltpu.async_copy(x_hbm.at[pl.ds(nbase, C)], idx_v.at[nb],
                                 sem_idx.at[nb])

            pltpu.make_async_copy(x_hbm.at[pl.ds(base0, C)], idx_v.at[b],
                                  sem_idx.at[b]).wait()

            @pl.when(c >= 2)
            def _drain():
                obase = base0 + (c - 2) * C
                pltpu.make_async_copy(rows_v.at[b],
                                      out_hbm.at[pl.ds(obase, C)],
                                      sem_out.at[b]).wait()

            pltpu.async_copy(table_s.at[idx_v.at[b]], rows_v.at[b],
                             sem_g.at[b]).wait()

            obase = base0 + c * C
            pltpu.async_copy(rows_v.at[b], out_hbm.at[pl.ds(obase, C)],
                             sem_out.at[b])

        @pl.loop(0, 2)
        def _tail(t):
            c = chunks - 2 + t
            b = c % 2
            obase = base0 + c * C
            pltpu.make_async_copy(rows_v.at[b], out_hbm.at[pl.ds(obase, C)],
                                  sem_out.at[b]).wait()

    return sc_lookup


def kernel(x, table):
    lead = x.shape[:-1]
    k = x.shape[-1]
    B = 1
    for s in x.shape:
        B *= s
    xf = x.reshape(B).astype(jnp.int32)
    out = _make_sc_lookup(B, 512)(xf, table)
    return out.reshape(lead + (k * EMBED,))
